# baseline (speedup 1.0000x reference)
import functools

import jax
import jax.numpy as jnp
from jax import lax
from jax.experimental import pallas as pl
from jax.experimental.pallas import tpu as pltpu
from jax.experimental.pallas import tpu_sc as plsc

NUM_EMB = 1_000_000
DIM = 32
SCALE = 1e-3

NUM_WORKERS = 32
LANES = 16

NP = 50
NS = 16384
UNIT = 512
NSUB = UNIT // 128
NUNITS = NP * NS // UNIT
PER_W = NUNITS // NUM_WORKERS
SB_PER_P = NS // UNIT


def _sc_body(idx_hbm, table_hbm, out_hbm, idx_all, rows, tbs,
             sg0, sg1, sw0, sw1):
    wid = lax.axis_index("s") * 2 + lax.axis_index("c")
    u0 = wid * PER_W
    scale_vec = jnp.full((LANES,), SCALE, dtype=jnp.float32)
    sg = (sg0, sg1)
    sw = (sw0, sw1)

    pltpu.sync_copy(idx_hbm.at[pl.ds(wid * PER_W * NSUB, PER_W * NSUB)],
                    idx_all)

    def start_gathers(k, b):
        for q in range(NSUB):
            pltpu.async_copy(
                table_hbm.at[idx_all.at[NSUB * k + q]], rows.at[b, q], sg[b])

    def wait_gathers(k, b):
        for q in range(NSUB):
            pltpu.make_async_copy(
                table_hbm.at[idx_all.at[NSUB * k + q]], rows.at[b, q],
                sg[b]).wait()

    iot = lax.iota(jnp.int32, LANES)
    hoff = jnp.full((LANES,), LANES, dtype=jnp.int32)

    def compute(b):
        @plsc.parallel_loop(0, 8 * LANES, 1, unroll=4)
        def zbody(z):
            ib = z // LANES
            t = z % LANES
            i0 = ib * LANES
            dd = (iot + jnp.full((LANES,), t, dtype=jnp.int32)) % LANES
            i0v = jnp.full((LANES,), i0, dtype=jnp.int32)
            iv = iot + i0v
            sv = (dd // 8) * 1024 + (dd % 8) * 128 + iv
            dhi = dd + hoff
            for q in range(NSUB):
                for h in range(2):
                    v = plsc.load_gather(
                        rows.at[b, q], [iv, dd if h == 0 else dhi])
                    plsc.store_scatter(
                        tbs[2 * q + h].at[b], [sv], v * scale_vec)

    def _write_slices(k, b):
        u = u0 + k
        p = u // SB_PER_P
        sb = u % SB_PER_P
        for q in range(NSUB):
            for h in range(2):
                for tih in range(2):
                    src = tbs[2 * q + h].at[b, pl.ds(tih * 1024, 1024)]
                    dst = out_hbm.at[p, 2 * h + tih,
                                     pl.ds(sb * 4096 + q * 1024, 1024)]
                    yield src, dst

    def start_write(k, b):
        for src, dst in _write_slices(k, b):
            pltpu.async_copy(src, dst, sw[b])

    def wait_write(k, b):
        for src, dst in _write_slices(k, b):
            pltpu.make_async_copy(src, dst, sw[b]).wait()

    start_gathers(0, 0)
    start_gathers(1, 1)
    for b in range(2):
        wait_gathers(b, b)
        compute(b)
        start_write(b, b)
        start_gathers(b + 2, b)

    def body(kk, carry):
        for b in range(2):
            k = 2 * kk + b
            wait_gathers(k, b)
            wait_write(k - 2, b)
            compute(b)
            start_write(k, b)
            start_gathers(k + 2, b)
        return carry
    lax.fori_loop(1, PER_W // 2 - 1, body, 0, unroll=False)

    for b in range(2):
        k = PER_W - 2 + b
        wait_gathers(k, b)
        wait_write(k - 2, b)
        compute(b)
        start_write(k, b)
    for b in range(2):
        wait_write(PER_W - 2 + b, b)


@functools.partial(
    pl.kernel,
    out_type=jax.ShapeDtypeStruct((NP, DIM // 8, NS * 8), jnp.float32),
    mesh=plsc.VectorSubcoreMesh(core_axis_name="c", subcore_axis_name="s"),
    scratch_types=[
        pltpu.VMEM((PER_W * NSUB, 128), jnp.int32),
        pltpu.VMEM((2, NSUB, 128, DIM), jnp.float32),
    ] + [pltpu.VMEM((2, 2048), jnp.float32) for _ in range(8)] + [
        pltpu.SemaphoreType.DMA,
        pltpu.SemaphoreType.DMA,
        pltpu.SemaphoreType.DMA,
        pltpu.SemaphoreType.DMA,
    ],
    compiler_params=pltpu.CompilerParams(
        use_tc_tiling_on_sc=False, needs_layout_passes=False),
)
def _gather_scaled(idx_hbm, table_hbm, out_hbm, idx_all, rows,
                   tb0, tb1, tb2, tb3, tb4, tb5, tb6, tb7,
                   sg0, sg1, sw0, sw1):
    _sc_body(idx_hbm, table_hbm, out_hbm, idx_all, rows,
             (tb0, tb1, tb2, tb3, tb4, tb5, tb6, tb7),
             sg0, sg1, sw0, sw1)


def kernel(inputs, weight):
    idx5 = inputs.T.reshape(NUNITS * NSUB, 128)
    p_out = _gather_scaled(idx5, weight)
    out = (p_out.reshape(NP, DIM // 8, NS // 128, 8, 128)
           .transpose(2, 4, 0, 1, 3)
           .reshape(NS, NP, DIM))
    return out


# speedup vs baseline: 1.0328x; 1.0328x over previous
import functools

import jax
import jax.numpy as jnp
from jax import lax
from jax.experimental import pallas as pl
from jax.experimental.pallas import tpu as pltpu
from jax.experimental.pallas import tpu_sc as plsc

NUM_EMB = 1_000_000
DIM = 32
SCALE = 1e-3

NUM_WORKERS = 32
LANES = 16

NP = 50
NS = 16384
UNIT = 512
NSUB = UNIT // 128
NUNITS = NP * NS // UNIT
PER_W = NUNITS // NUM_WORKERS
SB_PER_P = NS // UNIT


def _sc_body(idx_hbm, table_hbm, out_hbm, idx_all, rows, tbs,
             sg0, sg1, sw0, sw1):
    wid = lax.axis_index("s") * 2 + lax.axis_index("c")
    u0 = wid * PER_W
    scale_vec = jnp.full((LANES,), SCALE, dtype=jnp.float32)
    sg = (sg0, sg1)
    sw = (sw0, sw1)

    pltpu.sync_copy(idx_hbm.at[pl.ds(wid * PER_W * NSUB, PER_W * NSUB)],
                    idx_all)

    def start_gathers(k, b):
        for q in range(NSUB):
            pltpu.async_copy(
                table_hbm.at[idx_all.at[NSUB * k + q]], rows.at[b, q], sg[b])

    def wait_gathers(k, b):
        for q in range(NSUB):
            pltpu.make_async_copy(
                table_hbm.at[idx_all.at[NSUB * k + q]], rows.at[b, q],
                sg[b]).wait()

    iot = lax.iota(jnp.int32, LANES)
    hoff = jnp.full((LANES,), LANES, dtype=jnp.int32)

    def compute(b):
        @plsc.parallel_loop(0, 8 * LANES, 1, unroll=2)
        def zbody(z):
            ib = z // LANES
            t = z % LANES
            i0 = ib * LANES
            dd = (iot + jnp.full((LANES,), t, dtype=jnp.int32)) % LANES
            i0v = jnp.full((LANES,), i0, dtype=jnp.int32)
            iv = iot + i0v
            sv = (dd // 8) * 1024 + (dd % 8) * 128 + iv
            dhi = dd + hoff
            for q in range(NSUB):
                for h in range(2):
                    v = plsc.load_gather(
                        rows.at[b, q], [iv, dd if h == 0 else dhi])
                    plsc.store_scatter(
                        tbs[2 * q + h].at[b], [sv], v * scale_vec)

    def _write_slices(k, b):
        u = u0 + k
        p = u // SB_PER_P
        sb = u % SB_PER_P
        for q in range(NSUB):
            for h in range(2):
                for tih in range(2):
                    src = tbs[2 * q + h].at[b, pl.ds(tih * 1024, 1024)]
                    dst = out_hbm.at[p, 2 * h + tih,
                                     pl.ds(sb * 4096 + q * 1024, 1024)]
                    yield src, dst

    def start_write(k, b):
        for src, dst in _write_slices(k, b):
            pltpu.async_copy(src, dst, sw[b])

    def wait_write(k, b):
        for src, dst in _write_slices(k, b):
            pltpu.make_async_copy(src, dst, sw[b]).wait()

    start_gathers(0, 0)
    start_gathers(1, 1)
    for b in range(2):
        wait_gathers(b, b)
        compute(b)
        start_write(b, b)
        start_gathers(b + 2, b)

    def body(kk, carry):
        for b in range(2):
            k = 2 * kk + b
            wait_gathers(k, b)
            wait_write(k - 2, b)
            compute(b)
            start_write(k, b)
            start_gathers(k + 2, b)
        return carry
    lax.fori_loop(1, PER_W // 2 - 1, body, 0, unroll=False)

    for b in range(2):
        k = PER_W - 2 + b
        wait_gathers(k, b)
        wait_write(k - 2, b)
        compute(b)
        start_write(k, b)
    for b in range(2):
        wait_write(PER_W - 2 + b, b)


@functools.partial(
    pl.kernel,
    out_type=jax.ShapeDtypeStruct((NP, DIM // 8, NS * 8), jnp.float32),
    mesh=plsc.VectorSubcoreMesh(core_axis_name="c", subcore_axis_name="s"),
    scratch_types=[
        pltpu.VMEM((PER_W * NSUB, 128), jnp.int32),
        pltpu.VMEM((2, NSUB, 128, DIM), jnp.float32),
    ] + [pltpu.VMEM((2, 2048), jnp.float32) for _ in range(8)] + [
        pltpu.SemaphoreType.DMA,
        pltpu.SemaphoreType.DMA,
        pltpu.SemaphoreType.DMA,
        pltpu.SemaphoreType.DMA,
    ],
    compiler_params=pltpu.CompilerParams(
        use_tc_tiling_on_sc=False, needs_layout_passes=False),
)
def _gather_scaled(idx_hbm, table_hbm, out_hbm, idx_all, rows,
                   tb0, tb1, tb2, tb3, tb4, tb5, tb6, tb7,
                   sg0, sg1, sw0, sw1):
    _sc_body(idx_hbm, table_hbm, out_hbm, idx_all, rows,
             (tb0, tb1, tb2, tb3, tb4, tb5, tb6, tb7),
             sg0, sg1, sw0, sw1)


def kernel(inputs, weight):
    idx5 = inputs.T.reshape(NUNITS * NSUB, 128)
    p_out = _gather_scaled(idx5, weight)
    out = (p_out.reshape(NP, DIM // 8, NS // 128, 8, 128)
           .transpose(2, 4, 0, 1, 3)
           .reshape(NS, NP, DIM))
    return out


# speedup vs baseline: 1.0331x; 1.0002x over previous
import functools

import jax
import jax.numpy as jnp
from jax import lax
from jax.experimental import pallas as pl
from jax.experimental.pallas import tpu as pltpu
from jax.experimental.pallas import tpu_sc as plsc

NUM_EMB = 1_000_000
DIM = 32
SCALE = 1e-3

NUM_WORKERS = 32
LANES = 16

NP = 50
NS = 16384
UNIT = 512
NSUB = UNIT // 128
NUNITS = NP * NS // UNIT
PER_W = NUNITS // NUM_WORKERS
SB_PER_P = NS // UNIT


def _sc_body(idx_hbm, table_hbm, out_hbm, idx_all, rows, tbs,
             sg0, sg1, sw0, sw1):
    wid = lax.axis_index("s") * 2 + lax.axis_index("c")
    u0 = wid * PER_W
    scale_vec = jnp.full((LANES,), SCALE, dtype=jnp.float32)
    sg = (sg0, sg1)
    sw = (sw0, sw1)

    pltpu.sync_copy(idx_hbm.at[pl.ds(wid * PER_W * NSUB, PER_W * NSUB)],
                    idx_all)

    def start_gathers(k, b):
        for q in range(NSUB):
            pltpu.async_copy(
                table_hbm.at[idx_all.at[NSUB * k + q]], rows.at[b, q], sg[b])

    def wait_gathers(k, b):
        for q in range(NSUB):
            pltpu.make_async_copy(
                table_hbm.at[idx_all.at[NSUB * k + q]], rows.at[b, q],
                sg[b]).wait()

    iot = lax.iota(jnp.int32, LANES)
    hoff = jnp.full((LANES,), LANES, dtype=jnp.int32)

    def compute(b):
        @plsc.parallel_loop(0, 8 * LANES, 1, unroll=2)
        def zbody(z):
            ib = z // LANES
            t = z % LANES
            i0 = ib * LANES
            dd = (iot + jnp.full((LANES,), t, dtype=jnp.int32)) % LANES
            i0v = jnp.full((LANES,), i0, dtype=jnp.int32)
            iv = iot + i0v
            sv = (dd // 8) * 1024 + (dd % 8) * 128 + iv
            dhi = dd + hoff
            for q in range(NSUB):
                for h in range(2):
                    v = plsc.load_gather(
                        rows.at[b, q], [iv, dd if h == 0 else dhi])
                    plsc.store_scatter(
                        tbs[2 * q + h].at[b], [sv], v * scale_vec)

    def _write_slices(k, b):
        u = u0 + k
        p = u // SB_PER_P
        sb = u % SB_PER_P
        for q in range(NSUB):
            for h in range(2):
                for tih in range(2):
                    src = tbs[2 * q + h].at[b, pl.ds(tih * 1024, 1024)]
                    dst = out_hbm.at[p, 2 * h + tih,
                                     pl.ds(sb * 4096 + q * 1024, 1024)]
                    yield src, dst

    def start_write(k, b):
        for src, dst in _write_slices(k, b):
            pltpu.async_copy(src, dst, sw[b])

    def wait_write(k, b):
        for src, dst in _write_slices(k, b):
            pltpu.make_async_copy(src, dst, sw[b]).wait()

    start_gathers(0, 0)
    start_gathers(1, 1)
    for b in range(2):
        wait_gathers(b, b)
        compute(b)
        start_write(b, b)
        start_gathers(b + 2, b)

    def body(kk, carry):
        for b in range(2):
            k = 2 * kk + b
            wait_gathers(k, b)
            wait_write(k - 2, b)
            compute(b)
            start_write(k, b)
            start_gathers(k + 2, b)
        return carry
    lax.fori_loop(1, PER_W // 2 - 1, body, 0, unroll=False)

    for b in range(2):
        k = PER_W - 2 + b
        wait_gathers(k, b)
        wait_write(k - 2, b)
        compute(b)
        start_write(k, b)
    for b in range(2):
        wait_write(PER_W - 2 + b, b)


@functools.partial(
    pl.kernel,
    out_type=jax.ShapeDtypeStruct((NP, DIM // 8, NS * 8), jnp.float32),
    mesh=plsc.VectorSubcoreMesh(core_axis_name="c", subcore_axis_name="s"),
    scratch_types=[
        pltpu.VMEM((PER_W * NSUB, 128), jnp.int32),
        pltpu.VMEM((2, NSUB, 128, DIM), jnp.float32),
    ] + [pltpu.VMEM((2, 2048), jnp.float32) for _ in range(8)] + [
        pltpu.SemaphoreType.DMA,
        pltpu.SemaphoreType.DMA,
        pltpu.SemaphoreType.DMA,
        pltpu.SemaphoreType.DMA,
    ],
    compiler_params=pltpu.CompilerParams(
        use_tc_tiling_on_sc=False, needs_layout_passes=False),
)
def _gather_scaled(idx_hbm, table_hbm, out_hbm, idx_all, rows,
                   tb0, tb1, tb2, tb3, tb4, tb5, tb6, tb7,
                   sg0, sg1, sw0, sw1):
    _sc_body(idx_hbm, table_hbm, out_hbm, idx_all, rows,
             (tb0, tb1, tb2, tb3, tb4, tb5, tb6, tb7),
             sg0, sg1, sw0, sw1)


def kernel(inputs, weight):
    idx5 = jnp.minimum(inputs.T.reshape(NUNITS * NSUB, 128), NUM_EMB - 1)
    p_out = _gather_scaled(idx5, weight)
    out = (p_out.reshape(NP, DIM // 8, NS // 128, 8, 128)
           .transpose(2, 4, 0, 1, 3)
           .reshape(NS, NP, DIM))
    return out
